# baseline (speedup 1.0000x reference)
import functools

import jax
import jax.numpy as jnp
from jax import lax
from jax.experimental import pallas as pl
from jax.experimental.pallas import tpu as pltpu
from jax.experimental.pallas import tpu_sc as plsc

NC = 2
NS = 16
NW = NC * NS


@functools.cache
def _make_pad(b, h, g):
    return pl.pallas_call(
        lambda inp_ref, out_ref: out_ref.__setitem__(
            ..., jnp.pad(inp_ref[...], ((0, 0), (0, 128 - h)))
        ),
        grid=(b // g,),
        in_specs=[pl.BlockSpec((g, h), lambda i: (i, 0))],
        out_specs=pl.BlockSpec((g, 128), lambda i: (i, 0)),
        out_shape=jax.ShapeDtypeStruct((b, 128), jnp.int32),
    )


@functools.cache
def _make_unflatten(b, h, d, g):
    split = 128 // d
    rows = g * h * d // 128
    assert (g * h * d) % 128 == 0 and b % g == 0

    def body(in_ref, out_ref):
        x = in_ref[...]
        z = jnp.stack([x[:, j * d:(j + 1) * d] for j in range(split)], axis=1)
        out_ref[...] = z.reshape(g, h, d)

    return pl.pallas_call(
        body,
        grid=(b // g,),
        in_specs=[pl.BlockSpec((rows, 128), lambda i: (i, 0))],
        out_specs=pl.BlockSpec((g, h, d), lambda i: (i, 0, 0)),
        out_shape=jax.ShapeDtypeStruct((b, h, d), jnp.float32),
    )


@functools.cache
def _make_gather(b, h, v, d, chunk, nbuf):
    n = b * h
    b_per_w = b // NW
    n_per_w = b_per_w * h
    n_chunks = n_per_w // chunk
    n_groups = n_chunks // nbuf
    stage = b_per_w // 2
    idxf_len = n_per_w + 16
    assert n_per_w % chunk == 0 and n_chunks % nbuf == 0 and chunk % 8 == 0
    n_vl = h // 16 + (1 if h % 16 else 0)
    mesh = plsc.VectorSubcoreMesh(core_axis_name="c", subcore_axis_name="s")

    @functools.partial(
        pl.kernel,
        out_type=jax.ShapeDtypeStruct((n, d), jnp.float32),
        mesh=mesh,
        scratch_types=[
            pltpu.VMEM((stage, 128), jnp.int32),
            pltpu.VMEM((idxf_len,), jnp.int32),
            pltpu.VMEM((nbuf, chunk, d), jnp.float32),
            pltpu.SemaphoreType.DMA((nbuf,)),
            pltpu.SemaphoreType.DMA((nbuf,)),
        ],
        compiler_params=pltpu.CompilerParams(use_tc_tiling_on_sc=False),
    )
    def gather_kernel(table_hbm, idx_hbm, out_hbm, idx_pad, idxf, rows_v,
                      gsem, osem):
        wid = lax.axis_index("s") * NC + lax.axis_index("c")
        wb = pl.multiple_of(wid * b_per_w, b_per_w)
        base = pl.multiple_of(wid * n_per_w, chunk)

        for half in range(2):
            pltpu.sync_copy(idx_hbm.at[pl.ds(wb + half * stage, stage)],
                            idx_pad)

            def repack(i, carry, half=half):
                row = (half * stage + i) * h
                for j in range(n_vl):
                    idxf[pl.ds(row + j * 16, 16)] = idx_pad[i, pl.ds(j * 16, 16)]
                return carry

            lax.fori_loop(0, stage, repack, 0, unroll=False)

        def gather_desc(g, buf):
            off = pl.multiple_of(g * chunk, chunk)
            return pltpu.make_async_copy(
                table_hbm.at[idxf.at[pl.ds(off, chunk)]],
                rows_v.at[buf],
                gsem.at[buf],
            )

        def out_desc(g, buf):
            off = pl.multiple_of(base + g * chunk, chunk)
            return pltpu.make_async_copy(
                rows_v.at[buf],
                out_hbm.at[pl.ds(off, chunk)],
                osem.at[buf],
            )

        for buf in range(nbuf):
            gather_desc(jnp.int32(buf), buf).start()

        def group(go, carry):
            for buf in range(nbuf):
                g = go * nbuf + buf
                gather_desc(g, buf).wait()
                out_desc(g, buf).start()

            @pl.when(go < n_groups - 1)
            def _():
                for buf in range(nbuf):
                    out_desc(go * nbuf + buf, buf).wait()
                    gather_desc((go + 1) * nbuf + buf, buf).start()

            return carry

        lax.fori_loop(0, n_groups, group, 0, unroll=False)

        for buf in range(nbuf):
            out_desc(jnp.int32((n_groups - 1) * nbuf + buf), buf).wait()

    return gather_kernel


def kernel(model_input, table):
    b, h = model_input.shape
    v, d = table.shape
    n = b * h
    idx_pad = _make_pad(b, h, 512)(model_input)
    out_flat = _make_gather(b, h, v, d, 400, 4)(table, idx_pad)
    return _make_unflatten(b, h, d, 32)(out_flat.reshape(n * d // 128, 128))


# speedup vs baseline: 1.5060x; 1.5060x over previous
import functools

import jax
import jax.numpy as jnp
from jax import lax
from jax.experimental import pallas as pl
from jax.experimental.pallas import tpu as pltpu
from jax.experimental.pallas import tpu_sc as plsc

NC = 2
NS = 16
NW = NC * NS


@functools.cache
def _make_gather(b, h, v, d, nbuf):
    b_per_w = b // NW
    n_groups = b_per_w // nbuf
    assert b % NW == 0 and b_per_w % nbuf == 0
    mesh = plsc.VectorSubcoreMesh(core_axis_name="c", subcore_axis_name="s")

    @functools.partial(
        pl.kernel,
        out_type=jax.ShapeDtypeStruct((b, h, d), jnp.float32),
        mesh=mesh,
        scratch_types=[
            pltpu.VMEM((b_per_w, h), jnp.int32),
            pltpu.VMEM((nbuf, h, d), jnp.float32),
            pltpu.SemaphoreType.DMA((nbuf,)),
            pltpu.SemaphoreType.DMA((nbuf,)),
        ],
        compiler_params=pltpu.CompilerParams(use_tc_tiling_on_sc=False),
    )
    def gather_kernel(table_hbm, inp_hbm, out_hbm, idx_v, rows_v, gsem, osem):
        wid = lax.axis_index("s") * NC + lax.axis_index("c")
        wb = pl.multiple_of(wid * b_per_w, b_per_w)
        pltpu.sync_copy(inp_hbm.at[pl.ds(wb, b_per_w)], idx_v)

        def gather_desc(bi, buf):
            return pltpu.make_async_copy(
                table_hbm.at[idx_v.at[bi]],
                rows_v.at[buf],
                gsem.at[buf],
            )

        def out_desc(bi, buf):
            return pltpu.make_async_copy(
                rows_v.at[buf],
                out_hbm.at[wb + bi],
                osem.at[buf],
            )

        for buf in range(nbuf):
            gather_desc(jnp.int32(buf), buf).start()

        def group(go, carry):
            for buf in range(nbuf):
                bi = go * nbuf + buf
                gather_desc(bi, buf).wait()
                out_desc(bi, buf).start()

            @pl.when(go < n_groups - 1)
            def _():
                for buf in range(nbuf):
                    out_desc(go * nbuf + buf, buf).wait()
                    gather_desc((go + 1) * nbuf + buf, buf).start()

            return carry

        lax.fori_loop(0, n_groups, group, 0, unroll=False)

        for buf in range(nbuf):
            out_desc(jnp.int32((n_groups - 1) * nbuf + buf), buf).wait()

    return gather_kernel


def kernel(model_input, table):
    b, h = model_input.shape
    v, d = table.shape
    return _make_gather(b, h, v, d, 8)(table, model_input)


# speedup vs baseline: 1.5070x; 1.0006x over previous
import functools

import jax
import jax.numpy as jnp
from jax import lax
from jax.experimental import pallas as pl
from jax.experimental.pallas import tpu as pltpu
from jax.experimental.pallas import tpu_sc as plsc

NC = 2
NS = 16
NW = NC * NS


@functools.cache
def _make_gather(b, h, v, d, nbuf):
    b_per_w = b // NW
    n_groups = b_per_w // nbuf
    assert b % NW == 0 and b_per_w % nbuf == 0
    mesh = plsc.VectorSubcoreMesh(core_axis_name="c", subcore_axis_name="s")

    @functools.partial(
        pl.kernel,
        out_type=jax.ShapeDtypeStruct((b, h, d), jnp.float32),
        mesh=mesh,
        scratch_types=[
            pltpu.VMEM((b_per_w, h), jnp.int32),
            pltpu.VMEM((nbuf, h, d), jnp.float32),
            pltpu.SemaphoreType.DMA((nbuf,)),
            pltpu.SemaphoreType.DMA((nbuf,)),
        ],
        compiler_params=pltpu.CompilerParams(use_tc_tiling_on_sc=False),
    )
    def gather_kernel(table_hbm, inp_hbm, out_hbm, idx_v, rows_v, gsem, osem):
        wid = lax.axis_index("s") * NC + lax.axis_index("c")
        wb = pl.multiple_of(wid * b_per_w, b_per_w)
        pltpu.sync_copy(inp_hbm.at[pl.ds(wb, b_per_w)], idx_v)

        def gather_desc(bi, buf):
            return pltpu.make_async_copy(
                table_hbm.at[idx_v.at[bi]],
                rows_v.at[buf],
                gsem.at[buf],
            )

        def out_desc(bi, buf):
            return pltpu.make_async_copy(
                rows_v.at[buf],
                out_hbm.at[wb + bi],
                osem.at[buf],
            )

        for buf in range(nbuf):
            gather_desc(jnp.int32(buf), buf).start()

        def group(go, carry):
            for buf in range(nbuf):
                bi = go * nbuf + buf
                gather_desc(bi, buf).wait()
                out_desc(bi, buf).start()

            @pl.when(go < n_groups - 1)
            def _():
                for buf in range(nbuf):
                    out_desc(go * nbuf + buf, buf).wait()
                    gather_desc((go + 1) * nbuf + buf, buf).start()

            return carry

        lax.fori_loop(0, n_groups, group, 0, unroll=False)

        for buf in range(nbuf):
            out_desc(jnp.int32((n_groups - 1) * nbuf + buf), buf).wait()

    return gather_kernel


def kernel(model_input, table):
    b, h = model_input.shape
    v, d = table.shape
    idx = model_input.astype(jnp.int32)
    return _make_gather(b, h, v, d, 8)(table, idx)
